# Initial kernel scaffold; baseline (speedup 1.0000x reference)
import jax
import jax.numpy as jnp
from jax.experimental import pallas as pl


def kernel(input_ids, weight, lora_A, lora_B):
    raise NotImplementedError("write your pallas kernel here")


# speedup vs baseline: 5.5673x; 5.5673x over previous
import functools

import jax
import jax.numpy as jnp
from jax import lax
from jax.experimental import pallas as pl
from jax.experimental.pallas import tpu as pltpu
from jax.experimental.pallas import tpu_sc as plsc

_SCALING = 2.0

_NC = 2
_NS = 16
_NW = _NC * _NS

_CHUNK = 128


def _merge_body(w_ref, at_ref, b_ref, out_ref):
    delta = lax.dot_general(
        at_ref[...], b_ref[...],
        dimension_numbers=(((1,), (1,)), ((), ())),
        preferred_element_type=jnp.float32,
    )
    out_ref[...] = w_ref[...] + _SCALING * delta


def _build_merged(weight, lora_AT, lora_B):
    V, D = weight.shape
    R = lora_B.shape[1]
    BLK = 2000
    assert V % BLK == 0
    return pl.pallas_call(
        _merge_body,
        grid=(V // BLK,),
        in_specs=[
            pl.BlockSpec((BLK, D), lambda i: (i, 0)),
            pl.BlockSpec((BLK, R), lambda i: (i, 0)),
            pl.BlockSpec((D, R), lambda i: (0, 0)),
        ],
        out_specs=pl.BlockSpec((BLK, D), lambda i: (i, 0)),
        out_shape=jax.ShapeDtypeStruct((V, D), jnp.float32),
    )(weight, lora_AT, lora_B)


def _gather(merged, idx3):
    _, D = merged.shape
    nw, n_chunks, chunk = idx3.shape
    B = nw * n_chunks * chunk
    b_per_w = n_chunks * chunk
    mesh = plsc.VectorSubcoreMesh(core_axis_name="c", subcore_axis_name="s")

    @functools.partial(
        pl.kernel,
        mesh=mesh,
        out_type=jax.ShapeDtypeStruct((B, D), jnp.float32),
        compiler_params=pltpu.CompilerParams(use_tc_tiling_on_sc=False),
        scratch_types=[
            pltpu.VMEM((n_chunks, chunk), jnp.int32),
            pltpu.VMEM((chunk, D), jnp.float32),
            pltpu.SemaphoreType.DMA,
        ],
    )
    def k(table_hbm, idx_hbm, out_hbm, idx_v, rows_v, gsem):
        wid = lax.axis_index("s") * _NC + lax.axis_index("c")
        base = wid * b_per_w
        pltpu.sync_copy(idx_hbm.at[wid], idx_v)

        def body(g, _):
            pltpu.async_copy(merged_at(g), rows_v, gsem).wait()
            pltpu.sync_copy(rows_v, out_hbm.at[pl.ds(base + g * chunk, chunk)])
            return ()

        def merged_at(g):
            return table_hbm.at[idx_v.at[g]]

        lax.fori_loop(0, n_chunks, body, (), unroll=False)

    return k(merged, idx3)


def kernel(input_ids, weight, lora_A, lora_B):
    V, D = weight.shape
    merged = _build_merged(weight, lora_A.T, lora_B)
    B = input_ids.shape[0] * input_ids.shape[1]
    b_per_w = B // _NW
    n_chunks = b_per_w // _CHUNK
    idx3 = input_ids.reshape(_NW, n_chunks, _CHUNK).astype(jnp.int32)
    out = _gather(merged, idx3)
    return out.reshape(input_ids.shape[0], input_ids.shape[1], D)


# speedup vs baseline: 6.3387x; 1.1385x over previous
import functools

import jax
import jax.numpy as jnp
from jax import lax
from jax.experimental import pallas as pl
from jax.experimental.pallas import tpu as pltpu
from jax.experimental.pallas import tpu_sc as plsc

_SCALING = 2.0

_NC = 2
_NS = 16
_NW = _NC * _NS

_CHUNK = 128


_MERGE_BLK = 1024


def _merge_body(wt_ref, a_ref, bt_ref, out_ref):
    delta = lax.dot_general(
        a_ref[...], bt_ref[...],
        dimension_numbers=(((0,), (0,)), ((), ())),
        preferred_element_type=jnp.float32,
    )
    out_ref[...] = wt_ref[...].T + _SCALING * delta


def _build_merged(weightT, lora_A, lora_BT):
    D, V = weightT.shape
    R = lora_A.shape[0]
    BLK = _MERGE_BLK
    assert BLK % 128 == 0
    merged2 = pl.pallas_call(
        _merge_body,
        grid=(pl.cdiv(V, BLK),),
        in_specs=[
            pl.BlockSpec((D, BLK), lambda i: (0, i)),
            pl.BlockSpec((R, BLK), lambda i: (0, i)),
            pl.BlockSpec((R, D), lambda i: (0, 0)),
        ],
        out_specs=pl.BlockSpec((BLK, D), lambda i: (i, 0)),
        out_shape=jax.ShapeDtypeStruct((V, D), jnp.float32),
    )(weightT, lora_A, lora_BT)
    return merged2


def _gather(merged, idx3):
    _, D = merged.shape
    nw, n_chunks, chunk = idx3.shape
    B = nw * n_chunks * chunk
    b_per_w = n_chunks * chunk
    mesh = plsc.VectorSubcoreMesh(core_axis_name="c", subcore_axis_name="s")

    @functools.partial(
        pl.kernel,
        mesh=mesh,
        out_type=jax.ShapeDtypeStruct((B, D), jnp.float32),
        compiler_params=pltpu.CompilerParams(use_tc_tiling_on_sc=False),
        scratch_types=[
            pltpu.VMEM((n_chunks, chunk), jnp.int32),
            pltpu.VMEM((chunk, D), jnp.float32),
            pltpu.SemaphoreType.DMA,
        ],
    )
    def k(table_hbm, idx_hbm, out_hbm, idx_v, rows_v, gsem):
        wid = lax.axis_index("s") * _NC + lax.axis_index("c")
        base = wid * b_per_w
        pltpu.sync_copy(idx_hbm.at[wid], idx_v)

        def body(g, _):
            pltpu.async_copy(merged_at(g), rows_v, gsem).wait()
            pltpu.sync_copy(rows_v, out_hbm.at[pl.ds(base + g * chunk, chunk)])
            return ()

        def merged_at(g):
            return table_hbm.at[idx_v.at[g]]

        lax.fori_loop(0, n_chunks, body, (), unroll=False)

    return k(merged, idx3)


def kernel(input_ids, weight, lora_A, lora_B):
    V, D = weight.shape
    merged = _build_merged(weight.T, lora_A, lora_B.T)
    B = input_ids.shape[0] * input_ids.shape[1]
    b_per_w = B // _NW
    n_chunks = b_per_w // _CHUNK
    idx3 = input_ids.reshape(_NW, n_chunks, _CHUNK).astype(jnp.int32)
    out = _gather(merged, idx3)
    return out.reshape(input_ids.shape[0], input_ids.shape[1], D)


# speedup vs baseline: 8.3641x; 1.3195x over previous
import functools

import jax
import jax.numpy as jnp
from jax import lax
from jax.experimental import pallas as pl
from jax.experimental.pallas import tpu as pltpu
from jax.experimental.pallas import tpu_sc as plsc

_SCALING = 2.0

_NC = 2
_NS = 16
_NW = _NC * _NS

_CHUNK = 128
_MERGE_BLK = 1024


def _merge_body(wt_ref, a_ref, bt_ref, out_ref):
    delta = lax.dot_general(
        a_ref[...], bt_ref[...],
        dimension_numbers=(((0,), (0,)), ((), ())),
        preferred_element_type=jnp.float32,
    )
    out_ref[...] = wt_ref[...].T + _SCALING * delta


def _build_merged(weightT, lora_A, lora_BT):
    D, V = weightT.shape
    R = lora_A.shape[0]
    BLK = _MERGE_BLK
    return pl.pallas_call(
        _merge_body,
        grid=(pl.cdiv(V, BLK),),
        in_specs=[
            pl.BlockSpec((D, BLK), lambda i: (0, i)),
            pl.BlockSpec((R, BLK), lambda i: (0, i)),
            pl.BlockSpec((R, D), lambda i: (0, 0)),
        ],
        out_specs=pl.BlockSpec((BLK, D), lambda i: (i, 0)),
        out_shape=jax.ShapeDtypeStruct((V, D), jnp.float32),
    )(weightT, lora_A, lora_BT)


def _gather_scatter(merged, idx3, dest3):
    _, D = merged.shape
    nw, n_chunks, chunk = idx3.shape
    B = nw * n_chunks * chunk
    mesh = plsc.VectorSubcoreMesh(core_axis_name="c", subcore_axis_name="s")

    @functools.partial(
        pl.kernel,
        mesh=mesh,
        out_type=jax.ShapeDtypeStruct((B, D), jnp.float32),
        compiler_params=pltpu.CompilerParams(use_tc_tiling_on_sc=False),
        scratch_types=[
            pltpu.VMEM((n_chunks, chunk), jnp.int32),
            pltpu.VMEM((n_chunks, chunk), jnp.int32),
            pltpu.VMEM((chunk, D), jnp.float32),
            pltpu.SemaphoreType.DMA,
            pltpu.SemaphoreType.DMA,
        ],
    )
    def k(table_hbm, idx_hbm, dest_hbm, out_hbm, idx_v, dest_v, rows_v,
          gsem, ssem):
        wid = lax.axis_index("s") * _NC + lax.axis_index("c")
        pltpu.sync_copy(idx_hbm.at[wid], idx_v)
        pltpu.sync_copy(dest_hbm.at[wid], dest_v)

        def body(g, _):
            pltpu.async_copy(table_hbm.at[idx_v.at[g]], rows_v, gsem).wait()
            pltpu.async_copy(rows_v, out_hbm.at[dest_v.at[g]], ssem).wait()
            return ()

        lax.fori_loop(0, n_chunks, body, (), unroll=False)

    return k(merged, idx3, dest3)


def _transpose_body(in_ref, out_ref):
    x = in_ref[...]
    out_ref[...] = x.T.reshape(out_ref.shape)


def _to_output_layout(f3v, batch, seq, d):
    outT = pl.pallas_call(
        _transpose_body,
        grid=(seq // 2,),
        in_specs=[pl.BlockSpec((batch, 2 * d), lambda i: (i, 0))],
        out_specs=pl.BlockSpec((2, d, batch), lambda i: (i, 0, 0)),
        out_shape=jax.ShapeDtypeStruct((seq, d, batch), jnp.float32),
    )(f3v)
    return jnp.transpose(outT, (2, 0, 1))


def kernel(input_ids, weight, lora_A, lora_B):
    V, D = weight.shape
    batch, seq = input_ids.shape
    merged = _build_merged(weight.T, lora_A, lora_B.T)

    B = batch * seq
    n_chunks = B // (_NW * _CHUNK)
    idx3 = input_ids.reshape(_NW, n_chunks, _CHUNK).astype(jnp.int32)

    t = jnp.arange(B, dtype=jnp.int32)
    b, s = t // seq, t % seq
    dest3 = (((s // 2) * batch + b) * 2 + (s % 2)).reshape(_NW, n_chunks, _CHUNK)

    f3 = _gather_scatter(merged, idx3, dest3)
    f3v = f3.reshape(B // 2, 2 * D)
    return _to_output_layout(f3v, batch, seq, D)


# speedup vs baseline: 9.4109x; 1.1251x over previous
import functools

import jax
import jax.numpy as jnp
from jax import lax
from jax.experimental import pallas as pl
from jax.experimental.pallas import tpu as pltpu
from jax.experimental.pallas import tpu_sc as plsc

_SCALING = 2.0

_NC = 2
_NS = 16
_NW = _NC * _NS

_CHUNK = 128
_MERGE_BLK = 1024


def _merge_body(wt_ref, a_ref, bt_ref, out_ref):
    delta = lax.dot_general(
        a_ref[...], bt_ref[...],
        dimension_numbers=(((0,), (0,)), ((), ())),
        preferred_element_type=jnp.float32,
    )
    merged = wt_ref[...].T + _SCALING * delta
    half = merged.shape[0] // 2
    m3 = merged.reshape(half, 2, merged.shape[1])
    out_ref[...] = jnp.concatenate([m3[:, 0, :], m3[:, 1, :]], axis=1)


def _build_merged(weightT, lora_A, lora_BT):
    D, V = weightT.shape
    R = lora_A.shape[0]
    BLK = _MERGE_BLK
    merged2 = pl.pallas_call(
        _merge_body,
        grid=(pl.cdiv(V, BLK),),
        in_specs=[
            pl.BlockSpec((D, BLK), lambda i: (0, i)),
            pl.BlockSpec((R, BLK), lambda i: (0, i)),
            pl.BlockSpec((R, D), lambda i: (0, 0)),
        ],
        out_specs=pl.BlockSpec((BLK // 2, 2 * D), lambda i: (i, 0)),
        out_shape=jax.ShapeDtypeStruct((V // 2, 2 * D), jnp.float32),
    )(weightT, lora_A, lora_BT)
    return merged2.reshape(V, D)


def _gather_scatter(merged, idx3, dest3):
    _, D = merged.shape
    nw, n_chunks, chunk = idx3.shape
    B = nw * n_chunks * chunk
    mesh = plsc.VectorSubcoreMesh(core_axis_name="c", subcore_axis_name="s")

    @functools.partial(
        pl.kernel,
        mesh=mesh,
        out_type=jax.ShapeDtypeStruct((B, D), jnp.float32),
        compiler_params=pltpu.CompilerParams(use_tc_tiling_on_sc=False),
        scratch_types=[
            pltpu.VMEM((n_chunks, chunk), jnp.int32),
            pltpu.VMEM((n_chunks, chunk), jnp.int32),
            pltpu.VMEM((chunk, D), jnp.float32),
            pltpu.SemaphoreType.DMA,
            pltpu.SemaphoreType.DMA,
        ],
    )
    def k(table_hbm, idx_hbm, dest_hbm, out_hbm, idx_v, dest_v, rows_v,
          gsem, ssem):
        wid = lax.axis_index("s") * _NC + lax.axis_index("c")
        pltpu.sync_copy(idx_hbm.at[wid], idx_v)
        pltpu.sync_copy(dest_hbm.at[wid], dest_v)

        def body(g, _):
            pltpu.async_copy(table_hbm.at[idx_v.at[g]], rows_v, gsem).wait()
            pltpu.async_copy(rows_v, out_hbm.at[dest_v.at[g]], ssem).wait()
            return ()

        lax.fori_loop(0, n_chunks, body, (), unroll=False)

    return k(merged, idx3, dest3)


def _transpose_body(in_ref, out_ref):
    x = in_ref[...]
    out_ref[...] = x.T.reshape(out_ref.shape)


def _to_output_layout(f3v, batch, seq, d):
    outT = pl.pallas_call(
        _transpose_body,
        grid=(seq // 2,),
        in_specs=[pl.BlockSpec((batch, 2 * d), lambda i: (i, 0))],
        out_specs=pl.BlockSpec((2, d, batch), lambda i: (i, 0, 0)),
        out_shape=jax.ShapeDtypeStruct((seq, d, batch), jnp.float32),
    )(f3v)
    return jnp.transpose(outT, (2, 0, 1))


def kernel(input_ids, weight, lora_A, lora_B):
    V, D = weight.shape
    batch, seq = input_ids.shape
    merged = _build_merged(weight.T, lora_A, lora_B.T)

    B = batch * seq
    n_chunks = B // (_NW * _CHUNK)
    idx3 = input_ids.reshape(_NW, n_chunks, _CHUNK).astype(jnp.int32)

    t = jnp.arange(B, dtype=jnp.int32)
    b, s = t // seq, t % seq
    dest3 = (((s // 2) * batch + b) * 2 + (s % 2)).reshape(_NW, n_chunks, _CHUNK)

    f3 = _gather_scatter(merged, idx3, dest3)
    f3v = f3.reshape(B // 2, 2 * D)
    return _to_output_layout(f3v, batch, seq, D)


# speedup vs baseline: 9.9812x; 1.0606x over previous
import functools

import jax
import jax.numpy as jnp
from jax import lax
from jax.experimental import pallas as pl
from jax.experimental.pallas import tpu as pltpu
from jax.experimental.pallas import tpu_sc as plsc

_SCALING = 2.0

_NC = 2
_NS = 16
_NW = _NC * _NS

_CHUNK = 128
_MERGE_BLK = 1024


def _merge_body(wt_ref, a_ref, bt_ref, out_ref):
    delta = lax.dot_general(
        a_ref[...], bt_ref[...],
        dimension_numbers=(((0,), (0,)), ((), ())),
        preferred_element_type=jnp.float32,
    )
    merged = wt_ref[...].T + _SCALING * delta
    half = merged.shape[0] // 2
    m3 = merged.reshape(half, 2, merged.shape[1])
    out_ref[...] = jnp.concatenate([m3[:, 0, :], m3[:, 1, :]], axis=1)


def _build_merged(weightT, lora_A, lora_BT):
    D, V = weightT.shape
    R = lora_A.shape[0]
    BLK = _MERGE_BLK
    merged2 = pl.pallas_call(
        _merge_body,
        grid=(pl.cdiv(V, BLK),),
        in_specs=[
            pl.BlockSpec((D, BLK), lambda i: (0, i)),
            pl.BlockSpec((R, BLK), lambda i: (0, i)),
            pl.BlockSpec((R, D), lambda i: (0, 0)),
        ],
        out_specs=pl.BlockSpec((BLK // 2, 2 * D), lambda i: (i, 0)),
        out_shape=jax.ShapeDtypeStruct((V // 2, 2 * D), jnp.float32),
    )(weightT, lora_A, lora_BT)
    return merged2.reshape(V, D)


def _gather_scatter(merged, idx3, dest3):
    _, D = merged.shape
    nw, n_chunks, chunk = idx3.shape
    B = nw * n_chunks * chunk
    mesh = plsc.VectorSubcoreMesh(core_axis_name="c", subcore_axis_name="s")

    @functools.partial(
        pl.kernel,
        mesh=mesh,
        out_type=jax.ShapeDtypeStruct((B, D), jnp.float32),
        compiler_params=pltpu.CompilerParams(use_tc_tiling_on_sc=False),
        scratch_types=[
            pltpu.VMEM((n_chunks, chunk), jnp.int32),
            pltpu.VMEM((n_chunks, chunk), jnp.int32),
            pltpu.VMEM((chunk, D), jnp.float32),
            pltpu.VMEM((chunk, D), jnp.float32),
            pltpu.SemaphoreType.DMA,
            pltpu.SemaphoreType.DMA,
            pltpu.SemaphoreType.DMA,
            pltpu.SemaphoreType.DMA,
        ],
    )
    def k(table_hbm, idx_hbm, dest_hbm, out_hbm, idx_v, dest_v, rows0, rows1,
          gsem0, gsem1, ssem0, ssem1):
        wid = lax.axis_index("s") * _NC + lax.axis_index("c")
        pltpu.sync_copy(idx_hbm.at[wid], idx_v)
        pltpu.sync_copy(dest_hbm.at[wid], dest_v)

        def gath(g, rows, gsem):
            return pltpu.async_copy(table_hbm.at[idx_v.at[g]], rows, gsem)

        def scat(g, rows, ssem):
            return pltpu.async_copy(rows, out_hbm.at[dest_v.at[g]], ssem)

        def wait_gath(g, rows, gsem):
            pltpu.make_async_copy(table_hbm.at[idx_v.at[g]], rows, gsem).wait()

        def wait_scat(g, rows, ssem):
            pltpu.make_async_copy(rows, out_hbm.at[dest_v.at[g]], ssem).wait()

        gath(0, rows0, gsem0)

        def body(h, _):
            g = 2 * h
            wait_gath(g, rows0, gsem0)
            scat(g, rows0, ssem0)

            @pl.when(h > 0)
            def _():
                wait_scat(g - 1, rows1, ssem1)

            gath(g + 1, rows1, gsem1)
            wait_gath(g + 1, rows1, gsem1)
            scat(g + 1, rows1, ssem1)
            wait_scat(g, rows0, ssem0)

            @pl.when(h < n_chunks // 2 - 1)
            def _():
                gath(g + 2, rows0, gsem0)

            return ()

        lax.fori_loop(0, n_chunks // 2, body, (), unroll=False)
        wait_scat(n_chunks - 1, rows1, ssem1)

    return k(merged, idx3, dest3)


def _transpose_body(in_ref, out_ref):
    x = in_ref[...]
    out_ref[...] = x.T.reshape(out_ref.shape)


def _to_output_layout(f3v, batch, seq, d):
    outT = pl.pallas_call(
        _transpose_body,
        grid=(seq // 2,),
        in_specs=[pl.BlockSpec((batch, 2 * d), lambda i: (i, 0))],
        out_specs=pl.BlockSpec((2, d, batch), lambda i: (i, 0, 0)),
        out_shape=jax.ShapeDtypeStruct((seq, d, batch), jnp.float32),
    )(f3v)
    return jnp.transpose(outT, (2, 0, 1))


def kernel(input_ids, weight, lora_A, lora_B):
    V, D = weight.shape
    batch, seq = input_ids.shape
    merged = _build_merged(weight.T, lora_A, lora_B.T)

    B = batch * seq
    n_chunks = B // (_NW * _CHUNK)
    idx3 = input_ids.reshape(_NW, n_chunks, _CHUNK).astype(jnp.int32)

    t = jnp.arange(B, dtype=jnp.int32)
    b, s = t // seq, t % seq
    dest3 = (((s // 2) * batch + b) * 2 + (s % 2)).reshape(_NW, n_chunks, _CHUNK)

    f3 = _gather_scatter(merged, idx3, dest3)
    f3v = f3.reshape(B // 2, 2 * D)
    return _to_output_layout(f3v, batch, seq, D)


# speedup vs baseline: 12.1937x; 1.2217x over previous
import functools

import jax
import jax.numpy as jnp
from jax import lax
from jax.experimental import pallas as pl
from jax.experimental.pallas import tpu as pltpu
from jax.experimental.pallas import tpu_sc as plsc

_SCALING = 2.0

_NC = 2
_NS = 16
_NW = _NC * _NS

_CHUNK = 128
_MERGE_BLK = 2048


def _merge_body(wt_ref, a_ref, k_ref, out_ref):
    x = jnp.concatenate([wt_ref[...], a_ref[...]], axis=0)
    merged = lax.dot_general(
        x, k_ref[...],
        dimension_numbers=(((0,), (0,)), ((), ())),
        preferred_element_type=jnp.float32,
    )
    half = merged.shape[0] // 2
    out_ref[:, : merged.shape[1]] = merged[:half]
    out_ref[:, merged.shape[1]:] = merged[half:]


def _build_merged(weightT, lora_A, lora_BT):
    D, V = weightT.shape
    R = lora_A.shape[0]
    BLK = _MERGE_BLK
    nblk = pl.cdiv(V, BLK)
    v2 = nblk * BLK
    kmat = jnp.concatenate(
        [jnp.eye(D, dtype=jnp.float32), _SCALING * lora_BT], axis=0)
    merged2 = pl.pallas_call(
        _merge_body,
        grid=(nblk,),
        in_specs=[
            pl.BlockSpec((D, BLK), lambda i: (0, i)),
            pl.BlockSpec((R, BLK), lambda i: (0, i)),
            pl.BlockSpec((D + R, D), lambda i: (0, 0)),
        ],
        out_specs=pl.BlockSpec((BLK // 2, 2 * D), lambda i: (i, 0)),
        out_shape=jax.ShapeDtypeStruct((v2 // 2, 2 * D), jnp.float32),
    )(weightT, lora_A, kmat)
    return merged2.reshape(v2, D)


def _scramble(ids, blk):
    half = blk // 2
    i, r = ids // blk, ids % blk
    return i * blk + 2 * (r % half) + r // half


def _gather_scatter(merged, idx3, dest3):
    _, D = merged.shape
    nw, n_chunks, chunk = idx3.shape
    B = nw * n_chunks * chunk
    mesh = plsc.VectorSubcoreMesh(core_axis_name="c", subcore_axis_name="s")

    @functools.partial(
        pl.kernel,
        mesh=mesh,
        out_type=jax.ShapeDtypeStruct((B, D), jnp.float32),
        compiler_params=pltpu.CompilerParams(use_tc_tiling_on_sc=False),
        scratch_types=[
            pltpu.VMEM((n_chunks, chunk), jnp.int32),
            pltpu.VMEM((n_chunks, chunk), jnp.int32),
            pltpu.VMEM((chunk, D), jnp.float32),
            pltpu.VMEM((chunk, D), jnp.float32),
            pltpu.SemaphoreType.DMA,
            pltpu.SemaphoreType.DMA,
            pltpu.SemaphoreType.DMA,
            pltpu.SemaphoreType.DMA,
        ],
    )
    def k(table_hbm, idx_hbm, dest_hbm, out_hbm, idx_v, dest_v, rows0, rows1,
          gsem0, gsem1, ssem0, ssem1):
        wid = lax.axis_index("s") * _NC + lax.axis_index("c")
        pltpu.sync_copy(idx_hbm.at[wid], idx_v)
        pltpu.sync_copy(dest_hbm.at[wid], dest_v)

        def gath(g, rows, gsem):
            return pltpu.async_copy(table_hbm.at[idx_v.at[g]], rows, gsem)

        def scat(g, rows, ssem):
            return pltpu.async_copy(rows, out_hbm.at[dest_v.at[g]], ssem)

        def wait_gath(g, rows, gsem):
            pltpu.make_async_copy(table_hbm.at[idx_v.at[g]], rows, gsem).wait()

        def wait_scat(g, rows, ssem):
            pltpu.make_async_copy(rows, out_hbm.at[dest_v.at[g]], ssem).wait()

        gath(0, rows0, gsem0)

        def body(h, _):
            g = 2 * h
            wait_gath(g, rows0, gsem0)
            scat(g, rows0, ssem0)

            @pl.when(h > 0)
            def _():
                wait_scat(g - 1, rows1, ssem1)

            gath(g + 1, rows1, gsem1)
            wait_gath(g + 1, rows1, gsem1)
            scat(g + 1, rows1, ssem1)
            wait_scat(g, rows0, ssem0)

            @pl.when(h < n_chunks // 2 - 1)
            def _():
                gath(g + 2, rows0, gsem0)

            return ()

        lax.fori_loop(0, n_chunks // 2, body, (), unroll=False)
        wait_scat(n_chunks - 1, rows1, ssem1)

    return k(merged, idx3, dest3)


def _transpose_body(in_ref, out_ref):
    x = in_ref[...]
    out_ref[...] = x.T.reshape(out_ref.shape)


def _to_output_layout(f3v, batch, seq, d):
    outT = pl.pallas_call(
        _transpose_body,
        grid=(seq // 2,),
        in_specs=[pl.BlockSpec((batch, 2 * d), lambda i: (i, 0))],
        out_specs=pl.BlockSpec((2, d, batch), lambda i: (i, 0, 0)),
        out_shape=jax.ShapeDtypeStruct((seq, d, batch), jnp.float32),
    )(f3v)
    return jnp.transpose(outT, (2, 0, 1))


def kernel(input_ids, weight, lora_A, lora_B):
    V, D = weight.shape
    batch, seq = input_ids.shape
    merged = _build_merged(weight.T, lora_A, lora_B.T)

    B = batch * seq
    n_chunks = B // (_NW * _CHUNK)
    idx3 = _scramble(
        input_ids.reshape(_NW, n_chunks, _CHUNK).astype(jnp.int32), _MERGE_BLK)

    t = jnp.arange(B, dtype=jnp.int32)
    b, s = t // seq, t % seq
    dest3 = (((s // 2) * batch + b) * 2 + (s % 2)).reshape(_NW, n_chunks, _CHUNK)

    f3 = _gather_scatter(merged, idx3, dest3)
    f3v = f3.reshape(B // 2, 2 * D)
    return _to_output_layout(f3v, batch, seq, D)


# speedup vs baseline: 15.3206x; 1.2564x over previous
import functools

import jax
import jax.numpy as jnp
from jax import lax
from jax.experimental import pallas as pl
from jax.experimental.pallas import tpu as pltpu
from jax.experimental.pallas import tpu_sc as plsc

_SCALING = 2.0

_NC = 2
_NS = 16
_NW = _NC * _NS

_CHUNK = 128
_MERGE_BLK = 4096
_NBUF = 5
_LOOK = 3


def _merge_body(wt_ref, a_ref, k_ref, out_ref):
    x = jnp.concatenate([wt_ref[...], a_ref[...]], axis=0)
    merged = lax.dot_general(
        x, k_ref[...],
        dimension_numbers=(((0,), (0,)), ((), ())),
        preferred_element_type=jnp.float32,
    )
    half = merged.shape[0] // 2
    out_ref[:, : merged.shape[1]] = merged[:half]
    out_ref[:, merged.shape[1]:] = merged[half:]


def _build_merged(weightT, lora_A, lora_BT):
    D, V = weightT.shape
    R = lora_A.shape[0]
    BLK = _MERGE_BLK
    nblk = pl.cdiv(V, BLK)
    v2 = nblk * BLK
    kmat = jnp.concatenate(
        [jnp.eye(D, dtype=jnp.float32), _SCALING * lora_BT], axis=0)
    merged2 = pl.pallas_call(
        _merge_body,
        grid=(nblk,),
        in_specs=[
            pl.BlockSpec((D, BLK), lambda i: (0, i)),
            pl.BlockSpec((R, BLK), lambda i: (0, i)),
            pl.BlockSpec((D + R, D), lambda i: (0, 0)),
        ],
        out_specs=pl.BlockSpec((BLK // 2, 2 * D), lambda i: (i, 0)),
        out_shape=jax.ShapeDtypeStruct((v2 // 2, 2 * D), jnp.float32),
    )(weightT, lora_A, kmat)
    return merged2.reshape(v2, D)


def _scramble(ids, blk):
    half = blk // 2
    i, r = ids // blk, ids % blk
    return i * blk + 2 * (r % half) + r // half


def _gather_scatter(merged, idx3, dest3):
    _, D = merged.shape
    nw, n_chunks, chunk = idx3.shape
    B = nw * n_chunks * chunk
    mesh = plsc.VectorSubcoreMesh(core_axis_name="c", subcore_axis_name="s")

    @functools.partial(
        pl.kernel,
        mesh=mesh,
        out_type=jax.ShapeDtypeStruct((B, D), jnp.float32),
        compiler_params=pltpu.CompilerParams(use_tc_tiling_on_sc=False),
        scratch_types=(
            [pltpu.VMEM((n_chunks, chunk), jnp.int32),
             pltpu.VMEM((n_chunks, chunk), jnp.int32)]
            + [pltpu.VMEM((chunk, D), jnp.float32)] * _NBUF
            + [pltpu.SemaphoreType.DMA] * (2 * _NBUF)
        ),
    )
    def k(table_hbm, idx_hbm, dest_hbm, out_hbm, idx_v, dest_v, *bufs_sems):
        rows = bufs_sems[:_NBUF]
        gsem = bufs_sems[_NBUF:2 * _NBUF]
        ssem = bufs_sems[2 * _NBUF:]
        wid = lax.axis_index("s") * _NC + lax.axis_index("c")
        pltpu.sync_copy(idx_hbm.at[wid], idx_v)
        pltpu.sync_copy(dest_hbm.at[wid], dest_v)

        def gath(g, b):
            pltpu.async_copy(table_hbm.at[idx_v.at[g]], rows[b], gsem[b])

        def scat(g, b):
            pltpu.async_copy(rows[b], out_hbm.at[dest_v.at[g]], ssem[b])

        def wait_gath(g, b):
            pltpu.make_async_copy(
                table_hbm.at[idx_v.at[g]], rows[b], gsem[b]).wait()

        def wait_scat(g, b):
            pltpu.make_async_copy(
                rows[b], out_hbm.at[dest_v.at[g]], ssem[b]).wait()

        for b in range(_LOOK):
            gath(b, b)

        def body(h, _):
            g0 = h * _NBUF
            for b in range(_NBUF):
                g = g0 + b
                wait_gath(g, b)
                scat(g, b)

                @pl.when(g >= 2)
                def _(g=g, b=b):
                    wait_scat(g - 2, (b - 2) % _NBUF)

                @pl.when(g + _LOOK < n_chunks)
                def _(g=g, b=b):
                    gath(g + _LOOK, (b + _LOOK) % _NBUF)

            return ()

        lax.fori_loop(0, n_chunks // _NBUF, body, (), unroll=False)
        wait_scat(n_chunks - 2, (n_chunks - 2) % _NBUF)
        wait_scat(n_chunks - 1, (n_chunks - 1) % _NBUF)

    return k(merged, idx3, dest3)


def _transpose_body(in_ref, out_ref):
    x = in_ref[...]
    out_ref[...] = x.T.reshape(out_ref.shape)


def _to_output_layout(f3v, batch, seq, d):
    outT = pl.pallas_call(
        _transpose_body,
        grid=(seq // 2,),
        in_specs=[pl.BlockSpec((batch, 2 * d), lambda i: (i, 0))],
        out_specs=pl.BlockSpec((2, d, batch), lambda i: (i, 0, 0)),
        out_shape=jax.ShapeDtypeStruct((seq, d, batch), jnp.float32),
    )(f3v)
    return jnp.transpose(outT, (2, 0, 1))


def kernel(input_ids, weight, lora_A, lora_B):
    V, D = weight.shape
    batch, seq = input_ids.shape
    merged = _build_merged(weight.T, lora_A, lora_B.T)

    B = batch * seq
    n_chunks = B // (_NW * _CHUNK)
    idx3 = _scramble(
        input_ids.reshape(_NW, n_chunks, _CHUNK).astype(jnp.int32), _MERGE_BLK)

    t = jnp.arange(B, dtype=jnp.int32)
    b, s = t // seq, t % seq
    dest3 = (((s // 2) * batch + b) * 2 + (s % 2)).reshape(_NW, n_chunks, _CHUNK)

    f3 = _gather_scatter(merged, idx3, dest3)
    f3v = f3.reshape(B // 2, 2 * D)
    return _to_output_layout(f3v, batch, seq, D)


# speedup vs baseline: 15.9677x; 1.0422x over previous
import functools

import jax
import jax.numpy as jnp
from jax import lax
from jax.experimental import pallas as pl
from jax.experimental.pallas import tpu as pltpu
from jax.experimental.pallas import tpu_sc as plsc

_SCALING = 2.0

_NC = 2
_NS = 16
_NW = _NC * _NS

_CHUNK = 128
_MERGE_BLK = 4096
_NBUF = 5
_LOOK = 3


def _merge_body(wt_ref, a_ref, k_ref, out_ref):
    x = jnp.concatenate([wt_ref[...], a_ref[...]], axis=0)
    merged = lax.dot_general(
        x, k_ref[...],
        dimension_numbers=(((0,), (0,)), ((), ())),
        preferred_element_type=jnp.float32,
    )
    half = merged.shape[0] // 2
    out_ref[:, : merged.shape[1]] = merged[:half]
    out_ref[:, merged.shape[1]:] = merged[half:]


def _build_merged(weightT, lora_A, lora_BT):
    D, V = weightT.shape
    R = lora_A.shape[0]
    BLK = _MERGE_BLK
    nblk = pl.cdiv(V, BLK)
    v2 = nblk * BLK
    kmat = jnp.concatenate(
        [jnp.eye(D, dtype=jnp.float32), _SCALING * lora_BT], axis=0)
    merged2 = pl.pallas_call(
        _merge_body,
        grid=(nblk,),
        in_specs=[
            pl.BlockSpec((D, BLK), lambda i: (0, i)),
            pl.BlockSpec((R, BLK), lambda i: (0, i)),
            pl.BlockSpec((D + R, D), lambda i: (0, 0)),
        ],
        out_specs=pl.BlockSpec((BLK // 2, 2 * D), lambda i: (i, 0)),
        out_shape=jax.ShapeDtypeStruct((v2 // 2, 2 * D), jnp.float32),
    )(weightT, lora_A, kmat)
    return merged2.reshape(v2, D)


def _scramble(ids, blk):
    half = blk // 2
    i, r = ids // blk, ids % blk
    return i * blk + 2 * (r % half) + r // half


def _gather_scatter(merged, idxT):
    _, D = merged.shape
    n_chunks, batch = idxT.shape
    chunk = batch // _NW
    mesh = plsc.VectorSubcoreMesh(core_axis_name="c", subcore_axis_name="s")

    @functools.partial(
        pl.kernel,
        mesh=mesh,
        out_type=jax.ShapeDtypeStruct((n_chunks // 2, batch, 2, D),
                                      jnp.float32),
        compiler_params=pltpu.CompilerParams(use_tc_tiling_on_sc=False),
        scratch_types=(
            [pltpu.VMEM((n_chunks, chunk), jnp.int32)]
            + [pltpu.VMEM((chunk, D), jnp.float32)] * _NBUF
            + [pltpu.SemaphoreType.DMA] * (2 * _NBUF)
        ),
    )
    def k(table_hbm, idx_hbm, out_hbm, idx_v, *bufs_sems):
        rows = bufs_sems[:_NBUF]
        gsem = bufs_sems[_NBUF:2 * _NBUF]
        ssem = bufs_sems[2 * _NBUF:]
        wid = lax.axis_index("s") * _NC + lax.axis_index("c")
        b0 = wid * chunk
        pltpu.sync_copy(idx_hbm.at[:, pl.ds(b0, chunk)], idx_v)

        def gath(g, b):
            pltpu.async_copy(table_hbm.at[idx_v.at[g]], rows[b], gsem[b])

        def scat(g, b):
            pltpu.async_copy(
                rows[b], out_hbm.at[g // 2, pl.ds(b0, chunk), g % 2], ssem[b])

        def wait_gath(g, b):
            pltpu.make_async_copy(
                table_hbm.at[idx_v.at[g]], rows[b], gsem[b]).wait()

        def wait_scat(g, b):
            pltpu.make_async_copy(
                rows[b], out_hbm.at[g // 2, pl.ds(b0, chunk), g % 2],
                ssem[b]).wait()

        for b in range(_LOOK):
            gath(b, b)

        def body(h, _):
            g0 = h * _NBUF
            for b in range(_NBUF):
                g = g0 + b
                wait_gath(g, b)
                scat(g, b)

                @pl.when(g >= 2)
                def _(g=g, b=b):
                    wait_scat(g - 2, (b - 2) % _NBUF)

                @pl.when(g + _LOOK < n_chunks)
                def _(g=g, b=b):
                    gath(g + _LOOK, (b + _LOOK) % _NBUF)

            return ()

        lax.fori_loop(0, n_chunks // _NBUF, body, (), unroll=False)
        wait_scat(n_chunks - 2, (n_chunks - 2) % _NBUF)
        wait_scat(n_chunks - 1, (n_chunks - 1) % _NBUF)

    return k(merged, idxT)


def _transpose_body(in_ref, out_ref):
    x = in_ref[...]
    out_ref[...] = x.T.reshape(out_ref.shape)


def _to_output_layout(f3v, batch, seq, d):
    outT = pl.pallas_call(
        _transpose_body,
        grid=(seq // 2,),
        in_specs=[pl.BlockSpec((batch, 2 * d), lambda i: (i, 0))],
        out_specs=pl.BlockSpec((2, d, batch), lambda i: (i, 0, 0)),
        out_shape=jax.ShapeDtypeStruct((seq, d, batch), jnp.float32),
    )(f3v)
    return jnp.transpose(outT, (2, 0, 1))


def kernel(input_ids, weight, lora_A, lora_B):
    V, D = weight.shape
    batch, seq = input_ids.shape
    merged = _build_merged(weight.T, lora_A, lora_B.T)

    B = batch * seq
    idxT = _scramble(input_ids.T.astype(jnp.int32), _MERGE_BLK)
    f3 = _gather_scatter(merged, idxT)
    f3v = f3.reshape(B // 2, 2 * D)
    return _to_output_layout(f3v, batch, seq, D)
